# baseline (speedup 1.0000x reference)
import functools
import math
import numpy as np
import jax
import jax.numpy as jnp
from jax import lax
from jax.experimental import pallas as pl
from jax.experimental.pallas import tpu as pltpu
from jax.experimental.pallas import tpu_sc as plsc

_NLAT = 128
_NLON = 256
_LMAX = 50
_MMAX = 50
_NREF = 2048
_NQ = _NLAT * _NLON
_QT = 512
_NQT = _NQ // _QT


_SL = 8


def _knn_interp_body(qt_ref, qp_ref, rt_ref, rp_ref, rr_ref, out_ref):
    qt = qt_ref[0]
    qp = qp_ref[0]
    rt = rt_ref[0]
    rp = rp_ref[0]
    rr = rr_ref[0]

    bigf = jnp.float32(3.0e38)
    shp = (_SL, _QT)
    m1 = m2 = m3 = jnp.full(shp, bigf, jnp.float32)
    v1 = v2 = v3 = jnp.zeros(shp, jnp.float32)
    i1 = i2 = i3 = jnp.zeros(shp, jnp.float32)

    for s in range(_NREF // _SL):
        rts = rt[s * _SL:(s + 1) * _SL]
        rps = rp[s * _SL:(s + 1) * _SL]
        rrs = rr[s * _SL:(s + 1) * _SL]
        dt = rts - qt
        dp = rps - qp
        d2 = dt * dt + dp * dp
        sf = jnp.float32(s)
        b1 = d2 < m1
        b2 = d2 < m2
        b3 = d2 < m3
        m3 = jnp.where(b2, m2, jnp.where(b3, d2, m3))
        v3 = jnp.where(b2, v2, jnp.where(b3, rrs, v3))
        i3 = jnp.where(b2, i2, jnp.where(b3, sf, i3))
        m2 = jnp.where(b1, m1, jnp.where(b2, d2, m2))
        v2 = jnp.where(b1, v1, jnp.where(b2, rrs, v2))
        i2 = jnp.where(b1, i1, jnp.where(b2, sf, i2))
        m1 = jnp.where(b1, d2, m1)
        v1 = jnp.where(b1, rrs, v1)
        i1 = jnp.where(b1, sf, i1)

    rowio = jax.lax.broadcasted_iota(
        jnp.int32, shp, 0).astype(jnp.float32)
    slf = jnp.float32(_SL)
    ms = [m1, m2, m3]
    vs = [v1, v2, v3]
    gs = [i1 * slf + rowio, i2 * slf + rowio, i3 * slf + rowio]
    for k in range(3):
        mall = jnp.min(jnp.minimum(jnp.minimum(ms[0], ms[1]), ms[2]),
                       axis=0, keepdims=True)
        cs = [jnp.where(mq == mall, gq, bigf) for mq, gq in zip(ms, gs)]
        cmin = jnp.min(
            jnp.minimum(jnp.minimum(cs[0], cs[1]), cs[2]),
            axis=0, keepdims=True)
        hit = [gq == cmin for gq in gs]
        rk = jnp.sum(
            sum(jnp.where(h, vq, 0.0) for h, vq in zip(hit, vs)),
            axis=0, keepdims=True)
        out_ref[k, 0, 0, 0, :] = mall.reshape(_QT)
        out_ref[3 + k, 0, 0, 0, :] = rk.reshape(_QT)
        if k < 2:
            ms = [jnp.where(h, bigf, mq) for h, mq in zip(hit, ms)]


_NW = 32
_TCT = 44
_NQ_TC = _TCT * _QT
_NQ_SC = _NQ - _NQ_TC
_QPW = (2 * _NQ_SC) // _NW
_GRP = _QPW // 16


def _sc_knn_body(qt_hbm, qp_hbm, rt_hbm, rp_hbm, rr_hbm, out_hbm,
                 qt_v, qp_v, rt_v, rp_v, rr_v, out_v):
    wid = lax.axis_index("s") * 2 + lax.axis_index("c")
    b = wid // 16
    woff = (wid % 16) * _QPW
    qoff = _NQ_TC + woff

    pltpu.sync_copy(qt_hbm.at[pl.ds(qoff, _QPW)], qt_v)
    pltpu.sync_copy(qp_hbm.at[pl.ds(qoff, _QPW)], qp_v)
    pltpu.sync_copy(rt_hbm.at[pl.ds(b * _NREF, _NREF)], rt_v)
    pltpu.sync_copy(rp_hbm.at[pl.ds(b * _NREF, _NREF)], rp_v)
    pltpu.sync_copy(rr_hbm.at[pl.ds(b * _NREF, _NREF)], rr_v)

    big = jnp.float32(3.0e38)

    def group(g, carry):
        qtv = [qt_v[pl.ds(g * 64 + 16 * i, 16)] for i in range(4)]
        qpv = [qp_v[pl.ds(g * 64 + 16 * i, 16)] for i in range(4)]

        def body(jo, st):
            st = [list(st[6 * i: 6 * i + 6]) for i in range(4)]
            rtc = rt_v[pl.ds(jo * 16, 16)]
            rpc = rp_v[pl.ds(jo * 16, 16)]
            rrc = rr_v[pl.ds(jo * 16, 16)]
            for u in range(16):
                rts = rtc[u]
                rps = rpc[u]
                rrs = rrc[u]
                for i in range(4):
                    m1, m2, m3, v1, v2, v3 = st[i]
                    dt = qtv[i] - rts
                    dp = qpv[i] - rps
                    d2 = dt * dt + dp * dp
                    b1 = d2 < m1
                    b2 = d2 < m2
                    b3 = d2 < m3
                    st[i] = [
                        jnp.where(b1, d2, m1),
                        jnp.where(b1, m1, jnp.where(b2, d2, m2)),
                        jnp.where(b2, m2, jnp.where(b3, d2, m3)),
                        jnp.where(b1, rrs, v1),
                        jnp.where(b1, v1, jnp.where(b2, rrs, v2)),
                        jnp.where(b2, v2, jnp.where(b3, rrs, v3)),
                    ]
            return sum((tuple(s) for s in st), ())

        zf = jnp.zeros((16,), jnp.float32)
        bf = jnp.full((16,), big, jnp.float32)
        init = (bf, bf, bf, zf, zf, zf) * 4
        res = lax.fori_loop(0, _NREF // 16, body, init)
        for i in range(4):
            m1, m2, m3, v1, v2, v3 = res[6 * i: 6 * i + 6]
            for k, val in enumerate((m1, m2, m3, v1, v2, v3)):
                out_v[pl.ds(k * _QPW + g * 64 + i * 16, 16)] = val
        return carry

    lax.fori_loop(0, _GRP // 4, group, jnp.int32(0))
    for k in range(6):
        pltpu.sync_copy(
            out_v.at[pl.ds(k * _QPW, _QPW)],
            out_hbm.at[pl.ds(k * (2 * _NQ_SC) + b * _NQ_SC + woff, _QPW)])


def _sc_knn_call(qt, qp, rt, rp, rr):
    mesh = plsc.VectorSubcoreMesh(core_axis_name="c", subcore_axis_name="s")
    fn = functools.partial(
        pl.kernel,
        mesh=mesh,
        out_type=jax.ShapeDtypeStruct((6 * 2 * _NQ_SC,), jnp.float32),
        scratch_types=[
            pltpu.VMEM((_QPW,), jnp.float32),
            pltpu.VMEM((_QPW,), jnp.float32),
            pltpu.VMEM((_NREF,), jnp.float32),
            pltpu.VMEM((_NREF,), jnp.float32),
            pltpu.VMEM((_NREF,), jnp.float32),
            pltpu.VMEM((6 * _QPW,), jnp.float32),
        ],
    )(_sc_knn_body)
    return fn(qt, qp, rt, rp, rr)


def _sht_body(mv_ref, c_ref, w_ref, o_ref):
    zero = jnp.float32(0.0)
    d1 = jnp.sqrt(jnp.maximum(mv_ref[0, 0], zero))
    d2 = jnp.sqrt(jnp.maximum(mv_ref[1, 0], zero))
    d3 = jnp.sqrt(jnp.maximum(mv_ref[2, 0], zero))
    f = ((d1 * mv_ref[3, 0] + d2 * mv_ref[4, 0] + d3 * mv_ref[5, 0])
         / (d1 + d2 + d3))
    x = jnp.dot(f, c_ref[...], preferred_element_type=jnp.float32,
                precision=jax.lax.Precision.HIGHEST)
    t = w_ref[...] * x[:, None, :]
    o_ref[...] = jnp.sum(t, axis=0).reshape(1, _LMAX, _MMAX)


def _cos_matrix():
    n = np.arange(_NLON)[:, None].astype(np.float64)
    m = np.arange(_MMAX)[None, :].astype(np.float64)
    c = (2.0 * np.pi / _NLON) * np.cos(2.0 * np.pi * m * n / _NLON)
    return jnp.asarray(c.astype(np.float32))


def kernel(target, grid, sht_weights):
    x, y, z = target[..., 0], target[..., 1], target[..., 2]
    r = jnp.sqrt(x * x + y * y + z * z)
    theta = jnp.arccos(x / r)
    nzy = jnp.sqrt(z * z + y * y)
    a = jnp.arccos(y / nzy)
    phi = a + (2.0 * math.pi - 2.0 * a) * (z < 0).astype(jnp.float32)
    phi = phi - math.pi

    qth = grid[0, :, 0]
    qph = grid[0, :, 1]
    mv_sc = _sc_knn_call(qth, qph, theta.reshape(-1), phi.reshape(-1),
                         r.reshape(-1))

    qt = qth[: _NQ_TC].reshape(_TCT, 1, _QT)
    qp = qph[: _NQ_TC].reshape(_TCT, 1, _QT)
    rt3 = theta.reshape(2, _NREF, 1)
    rp3 = phi.reshape(2, _NREF, 1)
    rr3 = r.reshape(2, _NREF, 1)
    mv_tc = pl.pallas_call(
        _knn_interp_body,
        grid=(2, _TCT),
        in_specs=[
            pl.BlockSpec((1, 1, _QT), lambda b, t: (t, 0, 0)),
            pl.BlockSpec((1, 1, _QT), lambda b, t: (t, 0, 0)),
            pl.BlockSpec((1, _NREF, 1), lambda b, t: (b, 0, 0)),
            pl.BlockSpec((1, _NREF, 1), lambda b, t: (b, 0, 0)),
            pl.BlockSpec((1, _NREF, 1), lambda b, t: (b, 0, 0)),
        ],
        out_specs=pl.BlockSpec((6, 1, 1, 1, _QT),
                               lambda b, t: (0, b, t, 0, 0)),
        out_shape=jax.ShapeDtypeStruct((6, 2, _TCT, 1, _QT), jnp.float32),
    )(qt, qp, rt3, rp3, rr3)

    mv = jnp.concatenate(
        [mv_tc.reshape(6, 2, _NQ_TC), mv_sc.reshape(6, 2, _NQ_SC)], axis=2)
    mv4 = mv.reshape(6, 2, _NLAT, _NLON)
    cmat = _cos_matrix()
    w4 = jnp.transpose(sht_weights, (2, 1, 0))

    out = pl.pallas_call(
        _sht_body,
        grid=(2,),
        in_specs=[
            pl.BlockSpec((6, 1, _NLAT, _NLON), lambda b: (0, b, 0, 0)),
            pl.BlockSpec((_NLON, _MMAX), lambda b: (0, 0)),
            pl.BlockSpec((_NLAT, _LMAX, _MMAX), lambda b: (0, 0, 0)),
        ],
        out_specs=pl.BlockSpec((1, _LMAX, _MMAX), lambda b: (b, 0, 0)),
        out_shape=jax.ShapeDtypeStruct((2, _LMAX, _MMAX), jnp.float32),
    )(mv4, cmat, w4)
    return out


# speedup vs baseline: 1.1282x; 1.1282x over previous
import functools
import math
import numpy as np
import jax
import jax.numpy as jnp
from jax import lax
from jax.experimental import pallas as pl
from jax.experimental.pallas import tpu as pltpu
from jax.experimental.pallas import tpu_sc as plsc

_NLAT = 128
_NLON = 256
_LMAX = 50
_MMAX = 50
_NREF = 2048
_NQ = _NLAT * _NLON
_QT = 256
_NQT = _NQ // _QT


_SL = 8


def _knn_interp_body(qt_ref, qp_ref, rt_ref, rp_ref, rr_ref, out_ref):
    qt = qt_ref[0]
    qp = qp_ref[0]
    rt = rt_ref[0]
    rp = rp_ref[0]
    rr = rr_ref[0]

    bigf = jnp.float32(3.0e38)
    shp = (_SL, _QT)
    m1 = m2 = m3 = jnp.full(shp, bigf, jnp.float32)
    v1 = v2 = v3 = jnp.zeros(shp, jnp.float32)
    i1 = i2 = i3 = jnp.zeros(shp, jnp.float32)

    for s in range(_NREF // _SL):
        rts = rt[s * _SL:(s + 1) * _SL]
        rps = rp[s * _SL:(s + 1) * _SL]
        rrs = rr[s * _SL:(s + 1) * _SL]
        dt = rts - qt
        dp = rps - qp
        d2 = dt * dt + dp * dp
        sf = jnp.float32(s)
        b1 = d2 < m1
        b2 = d2 < m2
        b3 = d2 < m3
        m3 = jnp.where(b2, m2, jnp.where(b3, d2, m3))
        v3 = jnp.where(b2, v2, jnp.where(b3, rrs, v3))
        i3 = jnp.where(b2, i2, jnp.where(b3, sf, i3))
        m2 = jnp.where(b1, m1, jnp.where(b2, d2, m2))
        v2 = jnp.where(b1, v1, jnp.where(b2, rrs, v2))
        i2 = jnp.where(b1, i1, jnp.where(b2, sf, i2))
        m1 = jnp.where(b1, d2, m1)
        v1 = jnp.where(b1, rrs, v1)
        i1 = jnp.where(b1, sf, i1)

    rowio = jax.lax.broadcasted_iota(
        jnp.int32, shp, 0).astype(jnp.float32)
    slf = jnp.float32(_SL)
    ms = [m1, m2, m3]
    vs = [v1, v2, v3]
    gs = [i1 * slf + rowio, i2 * slf + rowio, i3 * slf + rowio]
    for k in range(3):
        mall = jnp.min(jnp.minimum(jnp.minimum(ms[0], ms[1]), ms[2]),
                       axis=0, keepdims=True)
        cs = [jnp.where(mq == mall, gq, bigf) for mq, gq in zip(ms, gs)]
        cmin = jnp.min(
            jnp.minimum(jnp.minimum(cs[0], cs[1]), cs[2]),
            axis=0, keepdims=True)
        hit = [gq == cmin for gq in gs]
        rk = jnp.sum(
            sum(jnp.where(h, vq, 0.0) for h, vq in zip(hit, vs)),
            axis=0, keepdims=True)
        out_ref[k, 0, 0, 0, :] = mall.reshape(_QT)
        out_ref[3 + k, 0, 0, 0, :] = rk.reshape(_QT)
        if k < 2:
            ms = [jnp.where(h, bigf, mq) for h, mq in zip(hit, ms)]


_NW = 32
_TCT = 88
_NQ_TC = _TCT * _QT
_NQ_SC = _NQ - _NQ_TC
_QPW = (2 * _NQ_SC) // _NW
_GRP = _QPW // 16


def _sc_knn_body(qt_hbm, qp_hbm, rt_hbm, rp_hbm, rr_hbm, out_hbm,
                 qt_v, qp_v, rt_v, rp_v, rr_v, out_v):
    wid = lax.axis_index("s") * 2 + lax.axis_index("c")
    b = wid // 16
    woff = (wid % 16) * _QPW
    qoff = _NQ_TC + woff

    pltpu.sync_copy(qt_hbm.at[pl.ds(qoff, _QPW)], qt_v)
    pltpu.sync_copy(qp_hbm.at[pl.ds(qoff, _QPW)], qp_v)
    pltpu.sync_copy(rt_hbm.at[pl.ds(b * _NREF, _NREF)], rt_v)
    pltpu.sync_copy(rp_hbm.at[pl.ds(b * _NREF, _NREF)], rp_v)
    pltpu.sync_copy(rr_hbm.at[pl.ds(b * _NREF, _NREF)], rr_v)

    big = jnp.float32(3.0e38)

    def group(g, carry):
        qtv = [qt_v[pl.ds(g * 64 + 16 * i, 16)] for i in range(4)]
        qpv = [qp_v[pl.ds(g * 64 + 16 * i, 16)] for i in range(4)]

        def body(jo, st):
            st = [list(st[6 * i: 6 * i + 6]) for i in range(4)]
            rtc = rt_v[pl.ds(jo * 16, 16)]
            rpc = rp_v[pl.ds(jo * 16, 16)]
            rrc = rr_v[pl.ds(jo * 16, 16)]
            for u in range(16):
                rts = rtc[u]
                rps = rpc[u]
                rrs = rrc[u]
                for i in range(4):
                    m1, m2, m3, v1, v2, v3 = st[i]
                    dt = qtv[i] - rts
                    dp = qpv[i] - rps
                    d2 = dt * dt + dp * dp
                    b1 = d2 < m1
                    b2 = d2 < m2
                    b3 = d2 < m3
                    st[i] = [
                        jnp.where(b1, d2, m1),
                        jnp.where(b1, m1, jnp.where(b2, d2, m2)),
                        jnp.where(b2, m2, jnp.where(b3, d2, m3)),
                        jnp.where(b1, rrs, v1),
                        jnp.where(b1, v1, jnp.where(b2, rrs, v2)),
                        jnp.where(b2, v2, jnp.where(b3, rrs, v3)),
                    ]
            return sum((tuple(s) for s in st), ())

        zf = jnp.zeros((16,), jnp.float32)
        bf = jnp.full((16,), big, jnp.float32)
        init = (bf, bf, bf, zf, zf, zf) * 4
        res = lax.fori_loop(0, _NREF // 16, body, init)
        for i in range(4):
            m1, m2, m3, v1, v2, v3 = res[6 * i: 6 * i + 6]
            for k, val in enumerate((m1, m2, m3, v1, v2, v3)):
                out_v[pl.ds(k * _QPW + g * 64 + i * 16, 16)] = val
        return carry

    lax.fori_loop(0, _GRP // 4, group, jnp.int32(0))
    for k in range(6):
        pltpu.sync_copy(
            out_v.at[pl.ds(k * _QPW, _QPW)],
            out_hbm.at[pl.ds(k * (2 * _NQ_SC) + b * _NQ_SC + woff, _QPW)])


def _sc_knn_call(qt, qp, rt, rp, rr):
    mesh = plsc.VectorSubcoreMesh(core_axis_name="c", subcore_axis_name="s")
    fn = functools.partial(
        pl.kernel,
        mesh=mesh,
        out_type=jax.ShapeDtypeStruct((6 * 2 * _NQ_SC,), jnp.float32),
        scratch_types=[
            pltpu.VMEM((_QPW,), jnp.float32),
            pltpu.VMEM((_QPW,), jnp.float32),
            pltpu.VMEM((_NREF,), jnp.float32),
            pltpu.VMEM((_NREF,), jnp.float32),
            pltpu.VMEM((_NREF,), jnp.float32),
            pltpu.VMEM((6 * _QPW,), jnp.float32),
        ],
    )(_sc_knn_body)
    return fn(qt, qp, rt, rp, rr)


def _sht_body(mv_ref, c_ref, w_ref, o_ref):
    zero = jnp.float32(0.0)
    d1 = jnp.sqrt(jnp.maximum(mv_ref[0, 0], zero))
    d2 = jnp.sqrt(jnp.maximum(mv_ref[1, 0], zero))
    d3 = jnp.sqrt(jnp.maximum(mv_ref[2, 0], zero))
    f = ((d1 * mv_ref[3, 0] + d2 * mv_ref[4, 0] + d3 * mv_ref[5, 0])
         / (d1 + d2 + d3))
    x = jnp.dot(f, c_ref[...], preferred_element_type=jnp.float32,
                precision=jax.lax.Precision.HIGHEST)
    t = w_ref[...] * x[:, None, :]
    o_ref[...] = jnp.sum(t, axis=0).reshape(1, _LMAX, _MMAX)


def _cos_matrix():
    n = np.arange(_NLON)[:, None].astype(np.float64)
    m = np.arange(_MMAX)[None, :].astype(np.float64)
    c = (2.0 * np.pi / _NLON) * np.cos(2.0 * np.pi * m * n / _NLON)
    return jnp.asarray(c.astype(np.float32))


def kernel(target, grid, sht_weights):
    x, y, z = target[..., 0], target[..., 1], target[..., 2]
    r = jnp.sqrt(x * x + y * y + z * z)
    theta = jnp.arccos(x / r)
    nzy = jnp.sqrt(z * z + y * y)
    a = jnp.arccos(y / nzy)
    phi = a + (2.0 * math.pi - 2.0 * a) * (z < 0).astype(jnp.float32)
    phi = phi - math.pi

    qth = grid[0, :, 0]
    qph = grid[0, :, 1]
    mv_sc = _sc_knn_call(qth, qph, theta.reshape(-1), phi.reshape(-1),
                         r.reshape(-1))

    qt = qth[: _NQ_TC].reshape(_TCT, 1, _QT)
    qp = qph[: _NQ_TC].reshape(_TCT, 1, _QT)
    rt3 = theta.reshape(2, _NREF, 1)
    rp3 = phi.reshape(2, _NREF, 1)
    rr3 = r.reshape(2, _NREF, 1)
    mv_tc = pl.pallas_call(
        _knn_interp_body,
        grid=(2, _TCT),
        in_specs=[
            pl.BlockSpec((1, 1, _QT), lambda b, t: (t, 0, 0)),
            pl.BlockSpec((1, 1, _QT), lambda b, t: (t, 0, 0)),
            pl.BlockSpec((1, _NREF, 1), lambda b, t: (b, 0, 0)),
            pl.BlockSpec((1, _NREF, 1), lambda b, t: (b, 0, 0)),
            pl.BlockSpec((1, _NREF, 1), lambda b, t: (b, 0, 0)),
        ],
        out_specs=pl.BlockSpec((6, 1, 1, 1, _QT),
                               lambda b, t: (0, b, t, 0, 0)),
        out_shape=jax.ShapeDtypeStruct((6, 2, _TCT, 1, _QT), jnp.float32),
    )(qt, qp, rt3, rp3, rr3)

    mv = jnp.concatenate(
        [mv_tc.reshape(6, 2, _NQ_TC), mv_sc.reshape(6, 2, _NQ_SC)], axis=2)
    mv4 = mv.reshape(6, 2, _NLAT, _NLON)
    cmat = _cos_matrix()
    w4 = jnp.transpose(sht_weights, (2, 1, 0))

    out = pl.pallas_call(
        _sht_body,
        grid=(2,),
        in_specs=[
            pl.BlockSpec((6, 1, _NLAT, _NLON), lambda b: (0, b, 0, 0)),
            pl.BlockSpec((_NLON, _MMAX), lambda b: (0, 0)),
            pl.BlockSpec((_NLAT, _LMAX, _MMAX), lambda b: (0, 0, 0)),
        ],
        out_specs=pl.BlockSpec((1, _LMAX, _MMAX), lambda b: (b, 0, 0)),
        out_shape=jax.ShapeDtypeStruct((2, _LMAX, _MMAX), jnp.float32),
    )(mv4, cmat, w4)
    return out
